# baseline (speedup 1.0000x reference)
import functools

import jax
import jax.numpy as jnp
from jax import lax
from jax.experimental import pallas as pl
from jax.experimental.pallas import tpu as pltpu
from jax.experimental.pallas import tpu_sc as plsc

_D = 768
_TB = 256
_CH = 2048
_KBT = 8192
_COMMIT_W = 0.25


def _tile_argmin(rT, a2v, cb_ref, b2t_s, j, kbt):
    big = jnp.int32(2 ** 30)
    best_v = None
    best_a = None
    ch = min(_CH, kbt)
    for c in range(kbt // ch):
        cb = cb_ref[c * ch:(c + 1) * ch, :]
        b2c = b2t_s[c * ch:(c + 1) * ch, :]
        ab = lax.dot_general(cb, rT, (((1,), (0,)), ((), ())))
        d2 = jnp.maximum((b2c + a2v) - 2.0 * ab, 0.0)
        lm = jnp.min(d2, axis=0, keepdims=True)
        ii = lax.broadcasted_iota(jnp.int32, d2.shape, 0) + c * ch
        la = jnp.min(jnp.where(d2 == lm, ii, big), axis=0, keepdims=True)
        if best_v is None:
            best_v, best_a = lm, la
        else:
            better = lm < best_v
            best_a = jnp.where(better, la, best_a)
            best_v = jnp.where(better, lm, best_v)
    return best_v, best_a + j * kbt


def _merge_state(best_v, best_a, idx_ref, minv_s, mina_s, i, j):
    row = pl.ds(i, 1)

    @pl.when(j == 0)
    def _():
        minv_s[row, :] = best_v
        mina_s[row, :] = best_a

    @pl.when(j != 0)
    def _():
        better = best_v < minv_s[row, :]
        mina_s[row, :] = jnp.where(better, best_a, mina_s[row, :])
        minv_s[row, :] = jnp.where(better, best_v, minv_s[row, :])

    idx_ref[0, 0, :] = mina_s[row, :][0]


def _dist_body_first(r_ref, cb_ref, idx_ref,
                     rT_s, a2_s, b2t_s, minv_s, mina_s):
    j = pl.program_id(0)
    i = pl.program_id(1)
    kbt = cb_ref.shape[0]
    col = pl.ds(i * _TB, _TB)

    ch = min(_CH, kbt)

    @pl.when(i == 0)
    def _():
        for c in range(kbt // ch):
            cb = cb_ref[c * ch:(c + 1) * ch, :]
            b2t_s[c * ch:(c + 1) * ch, :] = jnp.sum(
                cb * cb, axis=1, keepdims=True)

    @pl.when(j == 0)
    def _():
        r = r_ref[...]
        rT_s[:, col] = r.T
        a2_s[0, col] = jnp.sum(r * r, axis=1)

    best_v, best_a = _tile_argmin(rT_s[:, col], a2_s[0, col][None, :],
                                  cb_ref, b2t_s, j, kbt)
    _merge_state(best_v, best_a, idx_ref, minv_s, mina_s, i, j)


def _dist_body_update(r_ref, q_ref, cb_ref, idx_ref, rout_ref, a2out_ref,
                      rT_s, a2_s, b2t_s, minv_s, mina_s):
    j = pl.program_id(0)
    i = pl.program_id(1)
    kbt = cb_ref.shape[0]
    col = pl.ds(i * _TB, _TB)

    ch = min(_CH, kbt)

    @pl.when(i == 0)
    def _():
        for c in range(kbt // ch):
            cb = cb_ref[c * ch:(c + 1) * ch, :]
            b2t_s[c * ch:(c + 1) * ch, :] = jnp.sum(
                cb * cb, axis=1, keepdims=True)

    @pl.when(j == 0)
    def _():
        r = r_ref[...]
        q = q_ref[...]
        qst = r + (q - r)
        rn = r - qst
        rout_ref[col, :] = rn
        rT_s[:, col] = rn.T
        a2_s[0, col] = jnp.sum(rn * rn, axis=1)

    a2out_ref[0, 0, :] = a2_s[0, col]
    best_v, best_a = _tile_argmin(rT_s[:, col], a2_s[0, col][None, :],
                                  cb_ref, b2t_s, j, kbt)
    _merge_state(best_v, best_a, idx_ref, minv_s, mina_s, i, j)


def _first_sweep_spec(shape_blk, nb):
    def imap(j, i):
        return (jnp.where(j == 0, i, 0),) + (0,) * (len(shape_blk) - 1)
    return pl.BlockSpec(shape_blk, imap)


def _dist_first(xf, cb, nb, k):
    kbt = min(k, _KBT)
    kt = k // kbt
    n = nb * _TB
    return pl.pallas_call(
        _dist_body_first,
        grid=(kt, nb),
        in_specs=[
            _first_sweep_spec((_TB, _D), nb),
            pl.BlockSpec((kbt, _D), lambda j, i: (j, 0)),
        ],
        out_specs=pl.BlockSpec((1, 1, _TB), lambda j, i: (i, 0, 0)),
        out_shape=jax.ShapeDtypeStruct((nb, 1, _TB), jnp.int32),
        scratch_shapes=[
            pltpu.VMEM((_D, n), jnp.float32),
            pltpu.VMEM((1, n), jnp.float32),
            pltpu.VMEM((kbt, 1), jnp.float32),
            pltpu.VMEM((nb, _TB), jnp.float32),
            pltpu.VMEM((nb, _TB), jnp.int32),
        ],
    )(xf, cb)


def _dist_update(r, q, cb, nb, k):
    kbt = min(k, _KBT)
    kt = k // kbt
    n = nb * _TB
    return pl.pallas_call(
        _dist_body_update,
        grid=(kt, nb),
        in_specs=[
            _first_sweep_spec((_TB, _D), nb),
            _first_sweep_spec((_TB, _D), nb),
            pl.BlockSpec((kbt, _D), lambda j, i: (j, 0)),
        ],
        out_specs=[
            pl.BlockSpec((1, 1, _TB), lambda j, i: (i, 0, 0)),
            pl.BlockSpec((n, _D), lambda j, i: (0, 0)),
            pl.BlockSpec((1, 1, _TB), lambda j, i: (i, 0, 0)),
        ],
        out_shape=[
            jax.ShapeDtypeStruct((nb, 1, _TB), jnp.int32),
            jax.ShapeDtypeStruct((n, _D), jnp.float32),
            jax.ShapeDtypeStruct((nb, 1, _TB), jnp.float32),
        ],
        scratch_shapes=[
            pltpu.VMEM((_D, n), jnp.float32),
            pltpu.VMEM((1, n), jnp.float32),
            pltpu.VMEM((kbt, 1), jnp.float32),
            pltpu.VMEM((nb, _TB), jnp.float32),
            pltpu.VMEM((nb, _TB), jnp.int32),
        ],
    )(r, q, cb)


def _final_body(x_ref, r_ref, q_ref, quant_ref, a2out_ref):
    x = x_ref[...]
    r = r_ref[...]
    q = q_ref[...]
    qst = r + (q - r)
    rn = r - qst
    quant_ref[...] = x - rn
    a2out_ref[0, 0, :] = jnp.sum(rn * rn, axis=1)


def _final(xf, r, q, nb):
    n = nb * _TB
    return pl.pallas_call(
        _final_body,
        grid=(nb,),
        in_specs=[
            pl.BlockSpec((_TB, _D), lambda i: (i, 0)),
            pl.BlockSpec((_TB, _D), lambda i: (i, 0)),
            pl.BlockSpec((_TB, _D), lambda i: (i, 0)),
        ],
        out_specs=[
            pl.BlockSpec((_TB, _D), lambda i: (i, 0)),
            pl.BlockSpec((1, 1, _TB), lambda i: (i, 0, 0)),
        ],
        out_shape=[
            jax.ShapeDtypeStruct((n, _D), jnp.float32),
            jax.ShapeDtypeStruct((nb, 1, _TB), jnp.float32),
        ],
    )(xf, r, q)


def _make_sc_gather(n_tokens):
    info = plsc.get_sparse_core_info()
    nw = info.num_cores * info.num_subcores
    bpw = n_tokens // nw
    mesh = plsc.VectorSubcoreMesh(core_axis_name="c", subcore_axis_name="s")

    def body(table_hbm, idx_hbm, out_hbm, idx_v, rows_v, sem):
        wid = lax.axis_index("s") * info.num_cores + lax.axis_index("c")
        base = wid * bpw
        pltpu.sync_copy(idx_hbm.at[pl.ds(base, bpw)], idx_v)
        pltpu.async_copy(table_hbm.at[idx_v], rows_v, sem).wait()
        pltpu.sync_copy(rows_v, out_hbm.at[pl.ds(base, bpw)])

    return functools.partial(
        pl.kernel,
        mesh=mesh,
        out_type=jax.ShapeDtypeStruct((n_tokens, _D), jnp.float32),
        scratch_types=[
            pltpu.VMEM((bpw,), jnp.int32),
            pltpu.VMEM((bpw, _D), jnp.float32),
            pltpu.SemaphoreType.DMA,
        ],
    )(body)


def kernel(x, codebook_0, codebook_1, codebook_2, codebook_3):
    codebooks = [codebook_0, codebook_1, codebook_2, codebook_3]
    b, t, d = x.shape
    n = b * t
    nb = n // _TB
    xf = x.reshape(n, d)

    sc_gather = _make_sc_gather(n)

    idx0 = _dist_first(xf, codebooks[0], nb, codebooks[0].shape[0])
    q = sc_gather(codebooks[0], idx0.reshape(n))

    indices = [idx0]
    a2_sums = []
    r = xf
    for l in (1, 2, 3):
        k = codebooks[l].shape[0]
        idx_l, r, a2_l = _dist_update(r, q, codebooks[l], nb, k)
        indices.append(idx_l)
        a2_sums.append(jnp.sum(a2_l))
        q = sc_gather(codebooks[l], idx_l.reshape(n))

    quant, a2_last = _final(xf, r, q, nb)
    a2_sums.append(jnp.sum(a2_last))

    total_commit = jnp.asarray(0.0, dtype=jnp.float32)
    scale = jnp.float32(_COMMIT_W / (n * d))
    for s in a2_sums:
        total_commit = total_commit + s * scale

    quantized = quant.reshape(b, t, d)
    all_indices = jnp.stack([ix.reshape(b, t) for ix in indices], axis=-1)
    return quantized, all_indices, total_commit


# speedup vs baseline: 1.0255x; 1.0255x over previous
import functools

import jax
import jax.numpy as jnp
from jax import lax
from jax.experimental import pallas as pl
from jax.experimental.pallas import tpu as pltpu
from jax.experimental.pallas import tpu_sc as plsc

_D = 768
_TB = 256
_CH = 1024
_KBT = 8192
_COMMIT_W = 0.25


def _tile_argmin(rT, a2v, cb_ref, b2t_s, j, kbt):
    big = jnp.int32(2 ** 30)
    best_v = None
    best_a = None
    ch = min(_CH, kbt)
    for c in range(kbt // ch):
        cb = cb_ref[c * ch:(c + 1) * ch, :]
        b2c = b2t_s[c * ch:(c + 1) * ch, :]
        ab = lax.dot_general(cb, rT, (((1,), (0,)), ((), ())))
        d2 = jnp.maximum((b2c + a2v) - 2.0 * ab, 0.0)
        lm = jnp.min(d2, axis=0, keepdims=True)
        ii = lax.broadcasted_iota(jnp.int32, d2.shape, 0) + c * ch
        la = jnp.min(jnp.where(d2 == lm, ii, big), axis=0, keepdims=True)
        if best_v is None:
            best_v, best_a = lm, la
        else:
            better = lm < best_v
            best_a = jnp.where(better, la, best_a)
            best_v = jnp.where(better, lm, best_v)
    return best_v, best_a + j * kbt


def _merge_state(best_v, best_a, idx_ref, minv_s, mina_s, i, j):
    row = pl.ds(i, 1)

    @pl.when(j == 0)
    def _():
        minv_s[row, :] = best_v
        mina_s[row, :] = best_a

    @pl.when(j != 0)
    def _():
        better = best_v < minv_s[row, :]
        mina_s[row, :] = jnp.where(better, best_a, mina_s[row, :])
        minv_s[row, :] = jnp.where(better, best_v, minv_s[row, :])

    idx_ref[0, 0, :] = mina_s[row, :][0]


def _dist_body_first(r_ref, cb_ref, idx_ref,
                     rT_s, a2_s, b2t_s, minv_s, mina_s):
    j = pl.program_id(0)
    i = pl.program_id(1)
    kbt = cb_ref.shape[0]
    col = pl.ds(i * _TB, _TB)

    ch = min(_CH, kbt)

    @pl.when(i == 0)
    def _():
        for c in range(kbt // ch):
            cb = cb_ref[c * ch:(c + 1) * ch, :]
            b2t_s[c * ch:(c + 1) * ch, :] = jnp.sum(
                cb * cb, axis=1, keepdims=True)

    @pl.when(j == 0)
    def _():
        r = r_ref[...]
        rT_s[:, col] = r.T
        a2_s[0, col] = jnp.sum(r * r, axis=1)

    best_v, best_a = _tile_argmin(rT_s[:, col], a2_s[0, col][None, :],
                                  cb_ref, b2t_s, j, kbt)
    _merge_state(best_v, best_a, idx_ref, minv_s, mina_s, i, j)


def _dist_body_update(r_ref, q_ref, cb_ref, idx_ref, rout_ref, a2out_ref,
                      rT_s, a2_s, b2t_s, minv_s, mina_s):
    j = pl.program_id(0)
    i = pl.program_id(1)
    kbt = cb_ref.shape[0]
    col = pl.ds(i * _TB, _TB)

    ch = min(_CH, kbt)

    @pl.when(i == 0)
    def _():
        for c in range(kbt // ch):
            cb = cb_ref[c * ch:(c + 1) * ch, :]
            b2t_s[c * ch:(c + 1) * ch, :] = jnp.sum(
                cb * cb, axis=1, keepdims=True)

    @pl.when(j == 0)
    def _():
        r = r_ref[...]
        q = q_ref[...]
        qst = r + (q - r)
        rn = r - qst
        rout_ref[col, :] = rn
        rT_s[:, col] = rn.T
        a2_s[0, col] = jnp.sum(rn * rn, axis=1)

    a2out_ref[0, 0, :] = a2_s[0, col]
    best_v, best_a = _tile_argmin(rT_s[:, col], a2_s[0, col][None, :],
                                  cb_ref, b2t_s, j, kbt)
    _merge_state(best_v, best_a, idx_ref, minv_s, mina_s, i, j)


def _first_sweep_spec(shape_blk, nb):
    def imap(j, i):
        return (jnp.where(j == 0, i, 0),) + (0,) * (len(shape_blk) - 1)
    return pl.BlockSpec(shape_blk, imap)


def _dist_first(xf, cb, nb, k):
    kbt = min(k, _KBT)
    kt = k // kbt
    n = nb * _TB
    return pl.pallas_call(
        _dist_body_first,
        grid=(kt, nb),
        in_specs=[
            _first_sweep_spec((_TB, _D), nb),
            pl.BlockSpec((kbt, _D), lambda j, i: (j, 0)),
        ],
        out_specs=pl.BlockSpec((1, 1, _TB), lambda j, i: (i, 0, 0)),
        out_shape=jax.ShapeDtypeStruct((nb, 1, _TB), jnp.int32),
        scratch_shapes=[
            pltpu.VMEM((_D, n), jnp.float32),
            pltpu.VMEM((1, n), jnp.float32),
            pltpu.VMEM((kbt, 1), jnp.float32),
            pltpu.VMEM((nb, _TB), jnp.float32),
            pltpu.VMEM((nb, _TB), jnp.int32),
        ],
    )(xf, cb)


def _dist_update(r, q, cb, nb, k):
    kbt = min(k, _KBT)
    kt = k // kbt
    n = nb * _TB
    return pl.pallas_call(
        _dist_body_update,
        grid=(kt, nb),
        in_specs=[
            _first_sweep_spec((_TB, _D), nb),
            _first_sweep_spec((_TB, _D), nb),
            pl.BlockSpec((kbt, _D), lambda j, i: (j, 0)),
        ],
        out_specs=[
            pl.BlockSpec((1, 1, _TB), lambda j, i: (i, 0, 0)),
            pl.BlockSpec((n, _D), lambda j, i: (0, 0)),
            pl.BlockSpec((1, 1, _TB), lambda j, i: (i, 0, 0)),
        ],
        out_shape=[
            jax.ShapeDtypeStruct((nb, 1, _TB), jnp.int32),
            jax.ShapeDtypeStruct((n, _D), jnp.float32),
            jax.ShapeDtypeStruct((nb, 1, _TB), jnp.float32),
        ],
        scratch_shapes=[
            pltpu.VMEM((_D, n), jnp.float32),
            pltpu.VMEM((1, n), jnp.float32),
            pltpu.VMEM((kbt, 1), jnp.float32),
            pltpu.VMEM((nb, _TB), jnp.float32),
            pltpu.VMEM((nb, _TB), jnp.int32),
        ],
    )(r, q, cb)


def _final_body(x_ref, r_ref, q_ref, quant_ref, a2out_ref):
    x = x_ref[...]
    r = r_ref[...]
    q = q_ref[...]
    qst = r + (q - r)
    rn = r - qst
    quant_ref[...] = x - rn
    a2out_ref[0, 0, :] = jnp.sum(rn * rn, axis=1)


def _final(xf, r, q, nb):
    n = nb * _TB
    return pl.pallas_call(
        _final_body,
        grid=(nb,),
        in_specs=[
            pl.BlockSpec((_TB, _D), lambda i: (i, 0)),
            pl.BlockSpec((_TB, _D), lambda i: (i, 0)),
            pl.BlockSpec((_TB, _D), lambda i: (i, 0)),
        ],
        out_specs=[
            pl.BlockSpec((_TB, _D), lambda i: (i, 0)),
            pl.BlockSpec((1, 1, _TB), lambda i: (i, 0, 0)),
        ],
        out_shape=[
            jax.ShapeDtypeStruct((n, _D), jnp.float32),
            jax.ShapeDtypeStruct((nb, 1, _TB), jnp.float32),
        ],
    )(xf, r, q)


def _make_sc_gather(n_tokens):
    info = plsc.get_sparse_core_info()
    nw = info.num_cores * info.num_subcores
    bpw = n_tokens // nw
    mesh = plsc.VectorSubcoreMesh(core_axis_name="c", subcore_axis_name="s")

    def body(table_hbm, idx_hbm, out_hbm, idx_v, rows_v, sem):
        wid = lax.axis_index("s") * info.num_cores + lax.axis_index("c")
        base = wid * bpw
        pltpu.sync_copy(idx_hbm.at[pl.ds(base, bpw)], idx_v)
        pltpu.async_copy(table_hbm.at[idx_v], rows_v, sem).wait()
        pltpu.sync_copy(rows_v, out_hbm.at[pl.ds(base, bpw)])

    return functools.partial(
        pl.kernel,
        mesh=mesh,
        out_type=jax.ShapeDtypeStruct((n_tokens, _D), jnp.float32),
        scratch_types=[
            pltpu.VMEM((bpw,), jnp.int32),
            pltpu.VMEM((bpw, _D), jnp.float32),
            pltpu.SemaphoreType.DMA,
        ],
    )(body)


def kernel(x, codebook_0, codebook_1, codebook_2, codebook_3):
    codebooks = [codebook_0, codebook_1, codebook_2, codebook_3]
    b, t, d = x.shape
    n = b * t
    nb = n // _TB
    xf = x.reshape(n, d)

    sc_gather = _make_sc_gather(n)

    idx0 = _dist_first(xf, codebooks[0], nb, codebooks[0].shape[0])
    q = sc_gather(codebooks[0], idx0.reshape(n))

    indices = [idx0]
    a2_sums = []
    r = xf
    for l in (1, 2, 3):
        k = codebooks[l].shape[0]
        idx_l, r, a2_l = _dist_update(r, q, codebooks[l], nb, k)
        indices.append(idx_l)
        a2_sums.append(jnp.sum(a2_l))
        q = sc_gather(codebooks[l], idx_l.reshape(n))

    quant, a2_last = _final(xf, r, q, nb)
    a2_sums.append(jnp.sum(a2_last))

    total_commit = jnp.asarray(0.0, dtype=jnp.float32)
    scale = jnp.float32(_COMMIT_W / (n * d))
    for s in a2_sums:
        total_commit = total_commit + s * scale

    quantized = quant.reshape(b, t, d)
    all_indices = jnp.stack([ix.reshape(b, t) for ix in indices], axis=-1)
    return quantized, all_indices, total_commit


# speedup vs baseline: 1.0915x; 1.0644x over previous
import functools

import jax
import jax.numpy as jnp
from jax import lax
from jax.experimental import pallas as pl
from jax.experimental.pallas import tpu as pltpu
from jax.experimental.pallas import tpu_sc as plsc

_D = 768
_TB = 512
_CH = 1024
_KBT = 8192
_COMMIT_W = 0.25


def _tile_argmin(rT, a2v, cb_ref, b2t_s, j, kbt):
    big = jnp.int32(2 ** 30)
    best_v = None
    best_a = None
    ch = min(_CH, kbt)
    for c in range(kbt // ch):
        cb = cb_ref[c * ch:(c + 1) * ch, :]
        b2c = b2t_s[c * ch:(c + 1) * ch, :]
        ab = lax.dot_general(cb, rT, (((1,), (0,)), ((), ())))
        d2 = jnp.maximum((b2c + a2v) - 2.0 * ab, 0.0)
        lm = jnp.min(d2, axis=0, keepdims=True)
        ii = lax.broadcasted_iota(jnp.int32, d2.shape, 0) + c * ch
        la = jnp.min(jnp.where(d2 == lm, ii, big), axis=0, keepdims=True)
        if best_v is None:
            best_v, best_a = lm, la
        else:
            better = lm < best_v
            best_a = jnp.where(better, la, best_a)
            best_v = jnp.where(better, lm, best_v)
    return best_v, best_a + j * kbt


def _merge_state(best_v, best_a, idx_ref, minv_s, mina_s, i, j):
    row = pl.ds(i, 1)

    @pl.when(j == 0)
    def _():
        minv_s[row, :] = best_v
        mina_s[row, :] = best_a

    @pl.when(j != 0)
    def _():
        better = best_v < minv_s[row, :]
        mina_s[row, :] = jnp.where(better, best_a, mina_s[row, :])
        minv_s[row, :] = jnp.where(better, best_v, minv_s[row, :])

    idx_ref[0, 0, :] = mina_s[row, :][0]


def _dist_body_first(r_ref, cb_ref, idx_ref,
                     rT_s, a2_s, b2t_s, minv_s, mina_s):
    j = pl.program_id(0)
    i = pl.program_id(1)
    kbt = cb_ref.shape[0]
    col = pl.ds(i * _TB, _TB)

    ch = min(_CH, kbt)

    @pl.when(i == 0)
    def _():
        for c in range(kbt // ch):
            cb = cb_ref[c * ch:(c + 1) * ch, :]
            b2t_s[c * ch:(c + 1) * ch, :] = jnp.sum(
                cb * cb, axis=1, keepdims=True)

    @pl.when(j == 0)
    def _():
        r = r_ref[...]
        rT_s[:, col] = r.T
        a2_s[0, col] = jnp.sum(r * r, axis=1)

    best_v, best_a = _tile_argmin(rT_s[:, col], a2_s[0, col][None, :],
                                  cb_ref, b2t_s, j, kbt)
    _merge_state(best_v, best_a, idx_ref, minv_s, mina_s, i, j)


def _dist_body_update(r_ref, q_ref, cb_ref, idx_ref, rout_ref, a2out_ref,
                      rT_s, a2_s, b2t_s, minv_s, mina_s):
    j = pl.program_id(0)
    i = pl.program_id(1)
    kbt = cb_ref.shape[0]
    col = pl.ds(i * _TB, _TB)

    ch = min(_CH, kbt)

    @pl.when(i == 0)
    def _():
        for c in range(kbt // ch):
            cb = cb_ref[c * ch:(c + 1) * ch, :]
            b2t_s[c * ch:(c + 1) * ch, :] = jnp.sum(
                cb * cb, axis=1, keepdims=True)

    @pl.when(j == 0)
    def _():
        r = r_ref[...]
        q = q_ref[...]
        qst = r + (q - r)
        rn = r - qst
        rout_ref[col, :] = rn
        rT_s[:, col] = rn.T
        a2_s[0, col] = jnp.sum(rn * rn, axis=1)

    a2out_ref[0, 0, :] = a2_s[0, col]
    best_v, best_a = _tile_argmin(rT_s[:, col], a2_s[0, col][None, :],
                                  cb_ref, b2t_s, j, kbt)
    _merge_state(best_v, best_a, idx_ref, minv_s, mina_s, i, j)


def _first_sweep_spec(shape_blk, nb):
    def imap(j, i):
        return (jnp.where(j == 0, i, 0),) + (0,) * (len(shape_blk) - 1)
    return pl.BlockSpec(shape_blk, imap)


def _dist_first(xf, cb, nb, k):
    kbt = min(k, _KBT)
    kt = k // kbt
    n = nb * _TB
    return pl.pallas_call(
        _dist_body_first,
        grid=(kt, nb),
        in_specs=[
            _first_sweep_spec((_TB, _D), nb),
            pl.BlockSpec((kbt, _D), lambda j, i: (j, 0)),
        ],
        out_specs=pl.BlockSpec((1, 1, _TB), lambda j, i: (i, 0, 0)),
        out_shape=jax.ShapeDtypeStruct((nb, 1, _TB), jnp.int32),
        scratch_shapes=[
            pltpu.VMEM((_D, n), jnp.float32),
            pltpu.VMEM((1, n), jnp.float32),
            pltpu.VMEM((kbt, 1), jnp.float32),
            pltpu.VMEM((nb, _TB), jnp.float32),
            pltpu.VMEM((nb, _TB), jnp.int32),
        ],
    )(xf, cb)


def _dist_update(r, q, cb, nb, k):
    kbt = min(k, _KBT)
    kt = k // kbt
    n = nb * _TB
    return pl.pallas_call(
        _dist_body_update,
        grid=(kt, nb),
        in_specs=[
            _first_sweep_spec((_TB, _D), nb),
            _first_sweep_spec((_TB, _D), nb),
            pl.BlockSpec((kbt, _D), lambda j, i: (j, 0)),
        ],
        out_specs=[
            pl.BlockSpec((1, 1, _TB), lambda j, i: (i, 0, 0)),
            pl.BlockSpec((n, _D), lambda j, i: (0, 0)),
            pl.BlockSpec((1, 1, _TB), lambda j, i: (i, 0, 0)),
        ],
        out_shape=[
            jax.ShapeDtypeStruct((nb, 1, _TB), jnp.int32),
            jax.ShapeDtypeStruct((n, _D), jnp.float32),
            jax.ShapeDtypeStruct((nb, 1, _TB), jnp.float32),
        ],
        scratch_shapes=[
            pltpu.VMEM((_D, n), jnp.float32),
            pltpu.VMEM((1, n), jnp.float32),
            pltpu.VMEM((kbt, 1), jnp.float32),
            pltpu.VMEM((nb, _TB), jnp.float32),
            pltpu.VMEM((nb, _TB), jnp.int32),
        ],
    )(r, q, cb)


def _final_body(x_ref, r_ref, q_ref, quant_ref, a2out_ref):
    x = x_ref[...]
    r = r_ref[...]
    q = q_ref[...]
    qst = r + (q - r)
    rn = r - qst
    quant_ref[...] = x - rn
    a2out_ref[0, 0, :] = jnp.sum(rn * rn, axis=1)


def _final(xf, r, q, nb):
    n = nb * _TB
    return pl.pallas_call(
        _final_body,
        grid=(nb,),
        in_specs=[
            pl.BlockSpec((_TB, _D), lambda i: (i, 0)),
            pl.BlockSpec((_TB, _D), lambda i: (i, 0)),
            pl.BlockSpec((_TB, _D), lambda i: (i, 0)),
        ],
        out_specs=[
            pl.BlockSpec((_TB, _D), lambda i: (i, 0)),
            pl.BlockSpec((1, 1, _TB), lambda i: (i, 0, 0)),
        ],
        out_shape=[
            jax.ShapeDtypeStruct((n, _D), jnp.float32),
            jax.ShapeDtypeStruct((nb, 1, _TB), jnp.float32),
        ],
    )(xf, r, q)


def _make_sc_gather(n_tokens):
    info = plsc.get_sparse_core_info()
    nw = info.num_cores * info.num_subcores
    bpw = n_tokens // nw
    mesh = plsc.VectorSubcoreMesh(core_axis_name="c", subcore_axis_name="s")

    def body(table_hbm, idx_hbm, out_hbm, idx_v, rows_v, sem):
        wid = lax.axis_index("s") * info.num_cores + lax.axis_index("c")
        base = wid * bpw
        pltpu.sync_copy(idx_hbm.at[pl.ds(base, bpw)], idx_v)
        pltpu.async_copy(table_hbm.at[idx_v], rows_v, sem).wait()
        pltpu.sync_copy(rows_v, out_hbm.at[pl.ds(base, bpw)])

    return functools.partial(
        pl.kernel,
        mesh=mesh,
        out_type=jax.ShapeDtypeStruct((n_tokens, _D), jnp.float32),
        scratch_types=[
            pltpu.VMEM((bpw,), jnp.int32),
            pltpu.VMEM((bpw, _D), jnp.float32),
            pltpu.SemaphoreType.DMA,
        ],
    )(body)


def kernel(x, codebook_0, codebook_1, codebook_2, codebook_3):
    codebooks = [codebook_0, codebook_1, codebook_2, codebook_3]
    b, t, d = x.shape
    n = b * t
    nb = n // _TB
    xf = x.reshape(n, d)

    sc_gather = _make_sc_gather(n)

    idx0 = _dist_first(xf, codebooks[0], nb, codebooks[0].shape[0])
    q = sc_gather(codebooks[0], idx0.reshape(n))

    indices = [idx0]
    a2_sums = []
    r = xf
    for l in (1, 2, 3):
        k = codebooks[l].shape[0]
        idx_l, r, a2_l = _dist_update(r, q, codebooks[l], nb, k)
        indices.append(idx_l)
        a2_sums.append(jnp.sum(a2_l))
        q = sc_gather(codebooks[l], idx_l.reshape(n))

    quant, a2_last = _final(xf, r, q, nb)
    a2_sums.append(jnp.sum(a2_last))

    total_commit = jnp.asarray(0.0, dtype=jnp.float32)
    scale = jnp.float32(_COMMIT_W / (n * d))
    for s in a2_sums:
        total_commit = total_commit + s * scale

    quantized = quant.reshape(b, t, d)
    all_indices = jnp.stack([ix.reshape(b, t) for ix in indices], axis=-1)
    return quantized, all_indices, total_commit
